# Initial kernel scaffold; baseline (speedup 1.0000x reference)
import jax
import jax.numpy as jnp
from jax.experimental import pallas as pl


def kernel(x0, query, Wq, Wk, Wv, Wm, W1, W2, g1, b1, g2, b2):
    raise NotImplementedError("write your pallas kernel here")


# speedup vs baseline: 3.9037x; 3.9037x over previous
import functools
import math

import jax
import jax.numpy as jnp
from jax import lax
from jax.experimental import pallas as pl
from jax.experimental.pallas import tpu as pltpu
from jax.experimental.pallas import tpu_sc as plsc

B, L, C, N, H = 2, 4096, 256, 8, 8
DH = C // H
TOT = B * L
NC, NS, LANES = 2, 16, 16
NW = NC * NS
PER_W = TOT // NW
T = 16
CHUNKS = PER_W // T

_BLK = 512


def _dotT(a, w):
    return lax.dot_general(a, w, (((1,), (1,)), ((), ())),
                           preferred_element_type=jnp.float32)


def _ln(x, g, b, eps=1e-5):
    mu = jnp.mean(x, axis=-1, keepdims=True)
    xc = x - mu
    var = jnp.mean(xc * xc, axis=-1, keepdims=True)
    return xc * lax.rsqrt(var + eps) * g + b


def _qkv_body(x_ref, wq_ref, wk_ref, wv_ref, g_ref, b_ref, q_ref, k_ref, v_ref):
    xn = _ln(x_ref[...], g_ref[...], b_ref[...])
    q_ref[...] = _dotT(xn, wq_ref[...])
    k_ref[...] = _dotT(xn, wk_ref[...])
    v_ref[...] = _dotT(xn, wv_ref[...])


def _qkv_tc(x0f, Wq, Wk, Wv, g1, b1):
    grid = (TOT // _BLK,)
    row_spec = pl.BlockSpec((_BLK, C), lambda i: (i, 0))
    full_spec = pl.BlockSpec((C, C), lambda i: (0, 0))
    vec_spec = pl.BlockSpec((1, C), lambda i: (0, 0))
    return pl.pallas_call(
        _qkv_body,
        grid=grid,
        in_specs=[row_spec, full_spec, full_spec, full_spec, vec_spec, vec_spec],
        out_specs=[row_spec, row_spec, row_spec],
        out_shape=[jax.ShapeDtypeStruct((TOT, C), jnp.float32)] * 3,
    )(x0f, Wq, Wk, Wv, g1.reshape(1, C), b1.reshape(1, C))


def _post_body(x_ref, qv_ref, wm_ref, w1_ref, w2_ref, g_ref, b_ref, o_ref):
    msg = x_ref[...] + _dotT(qv_ref[...], wm_ref[...])
    hid = jnp.maximum(_dotT(msg, w1_ref[...]), 0.0)
    mlp = _dotT(hid, w2_ref[...])
    o_ref[...] = x_ref[...] + _ln(mlp, g_ref[...], b_ref[...])


def _post_tc(x0f, qvals, Wm, W1, W2, g2, b2):
    grid = (TOT // _BLK,)
    row_spec = pl.BlockSpec((_BLK, C), lambda i: (i, 0))
    full_spec = pl.BlockSpec((C, C), lambda i: (0, 0))
    vec_spec = pl.BlockSpec((1, C), lambda i: (0, 0))
    return pl.pallas_call(
        _post_body,
        grid=grid,
        in_specs=[row_spec, row_spec, full_spec, full_spec, full_spec,
                  vec_spec, vec_spec],
        out_specs=row_spec,
        out_shape=jax.ShapeDtypeStruct((TOT, C), jnp.float32),
    )(x0f, qvals, Wm, W1, W2, g2.reshape(1, C), b2.reshape(1, C))


_SCALE = 1.0 / math.sqrt(DH)


def _attn_sc_body(q_hbm, k_hbm, v_hbm, idx_hbm, out_hbm,
                  idxv, karr, varr, qarr, oarr, a_arr, sem_k, sem_v):
    wid = lax.axis_index("s") * NC + lax.axis_index("c")
    base = wid * PER_W
    iota = lax.iota(jnp.int32, LANES)

    def chunk_body(ci, carry):
        t0 = base + ci * T
        pltpu.sync_copy(idx_hbm.at[pl.ds(t0 * N, T * N)], idxv)
        cp_k = pltpu.async_copy(k_hbm.at[idxv], karr, sem_k)
        cp_v = pltpu.async_copy(v_hbm.at[idxv], varr, sem_v)
        pltpu.sync_copy(q_hbm.at[pl.ds(t0, T)], qarr)
        cp_k.wait()
        cp_v.wait()

        for h in range(H):
            def dbody(d, accs):
                col = jnp.full((LANES,), h * DH, jnp.int32) + d
                qv = plsc.load_gather(qarr, [iota, col])
                return [accs[n] + qv * plsc.load_gather(karr, [iota * N + n, col])
                        for n in range(N)]

            accs = lax.fori_loop(
                0, DH, dbody, [jnp.zeros((LANES,), jnp.float32)] * N)
            ps = [a * _SCALE for a in accs]
            m = ps[0]
            for n in range(1, N):
                m = jnp.maximum(m, ps[n])
            es = [jnp.exp(p - m) for p in ps]
            s = es[0]
            for n in range(1, N):
                s = s + es[n]
            for n in range(N):
                a_arr[h, n, :] = es[n] / s

        for h in range(H):
            avs = [a_arr[h, n, :] for n in range(N)]

            def cbody(c, carry2):
                col = jnp.full((LANES,), 0, jnp.int32) + c
                acc = jnp.zeros((LANES,), jnp.float32)
                for n in range(N):
                    acc = acc + avs[n] * plsc.load_gather(varr, [iota * N + n, col])
                plsc.store_scatter(oarr, [iota, col], acc)
                return carry2

            lax.fori_loop(h * DH, (h + 1) * DH, cbody, 0)

        pltpu.sync_copy(oarr, out_hbm.at[pl.ds(t0, T)])
        return carry

    lax.fori_loop(0, CHUNKS, chunk_body, 0)


def _attn_sc(q, k, v, qidx):
    mesh = plsc.VectorSubcoreMesh(core_axis_name="c", subcore_axis_name="s")
    fn = functools.partial(
        pl.kernel,
        mesh=mesh,
        out_type=jax.ShapeDtypeStruct((TOT, C), jnp.float32),
        scratch_types=[
            pltpu.VMEM((T * N,), jnp.int32),
            pltpu.VMEM((T * N, C), jnp.float32),
            pltpu.VMEM((T * N, C), jnp.float32),
            pltpu.VMEM((T, C), jnp.float32),
            pltpu.VMEM((T, C), jnp.float32),
            pltpu.VMEM((H, N, LANES), jnp.float32),
            pltpu.SemaphoreType.DMA,
            pltpu.SemaphoreType.DMA,
        ],
        compiler_params=pltpu.CompilerParams(use_tc_tiling_on_sc=False,
                                             needs_layout_passes=False),
    )(_attn_sc_body)
    return fn(q, k, v, qidx)


def kernel(x0, query, Wq, Wk, Wv, Wm, W1, W2, g1, b1, g2, b2):
    x0f = x0.reshape(TOT, C)
    qidx = (query.astype(jnp.int32)
            + (jnp.arange(B, dtype=jnp.int32) * L)[:, None, None]).reshape(-1)
    q, k, v = _qkv_tc(x0f, Wq, Wk, Wv, g1, b1)
    qvals = _attn_sc(q, k, v, qidx)
    out = _post_tc(x0f, qvals, Wm, W1, W2, g2, b2)
    return out.reshape(B, L, C)


# speedup vs baseline: 4.1109x; 1.0531x over previous
import functools
import math

import jax
import jax.numpy as jnp
from jax import lax
from jax.experimental import pallas as pl
from jax.experimental.pallas import tpu as pltpu
from jax.experimental.pallas import tpu_sc as plsc

B, L, C, N, H = 2, 4096, 256, 8, 8
DH = C // H
TOT = B * L
NC, NS, LANES = 2, 16, 16
NW = NC * NS
PER_W = TOT // NW
T = 16
CHUNKS = PER_W // T

_BLK = 512


def _dotT(a, w):
    return lax.dot_general(a, w, (((1,), (1,)), ((), ())),
                           preferred_element_type=jnp.float32)


def _ln(x, g, b, eps=1e-5):
    mu = jnp.mean(x, axis=-1, keepdims=True)
    xc = x - mu
    var = jnp.mean(xc * xc, axis=-1, keepdims=True)
    return xc * lax.rsqrt(var + eps) * g + b


def _qkv_body(x_ref, wq_ref, wk_ref, wv_ref, g_ref, b_ref, q_ref, k_ref, v_ref):
    xn = _ln(x_ref[...], g_ref[...], b_ref[...])
    q_ref[...] = _dotT(xn, wq_ref[...])
    k_ref[...] = _dotT(xn, wk_ref[...])
    v_ref[...] = _dotT(xn, wv_ref[...])


def _qkv_tc(x0f, Wq, Wk, Wv, g1, b1):
    grid = (TOT // _BLK,)
    row_spec = pl.BlockSpec((_BLK, C), lambda i: (i, 0))
    full_spec = pl.BlockSpec((C, C), lambda i: (0, 0))
    vec_spec = pl.BlockSpec((1, C), lambda i: (0, 0))
    return pl.pallas_call(
        _qkv_body,
        grid=grid,
        in_specs=[row_spec, full_spec, full_spec, full_spec, vec_spec, vec_spec],
        out_specs=[row_spec, row_spec, row_spec],
        out_shape=[jax.ShapeDtypeStruct((TOT, C), jnp.float32)] * 3,
    )(x0f, Wq, Wk, Wv, g1.reshape(1, C), b1.reshape(1, C))


def _post_body(x_ref, qv_ref, wm_ref, w1_ref, w2_ref, g_ref, b_ref, o_ref):
    msg = x_ref[...] + _dotT(qv_ref[...], wm_ref[...])
    hid = jnp.maximum(_dotT(msg, w1_ref[...]), 0.0)
    mlp = _dotT(hid, w2_ref[...])
    o_ref[...] = x_ref[...] + _ln(mlp, g_ref[...], b_ref[...])


def _post_tc(x0f, qvals, Wm, W1, W2, g2, b2):
    grid = (TOT // _BLK,)
    row_spec = pl.BlockSpec((_BLK, C), lambda i: (i, 0))
    full_spec = pl.BlockSpec((C, C), lambda i: (0, 0))
    vec_spec = pl.BlockSpec((1, C), lambda i: (0, 0))
    return pl.pallas_call(
        _post_body,
        grid=grid,
        in_specs=[row_spec, row_spec, full_spec, full_spec, full_spec,
                  vec_spec, vec_spec],
        out_specs=row_spec,
        out_shape=jax.ShapeDtypeStruct((TOT, C), jnp.float32),
    )(x0f, qvals, Wm, W1, W2, g2.reshape(1, C), b2.reshape(1, C))


_SCALE = 1.0 / math.sqrt(DH)


def _attn_sc_body(q_hbm, k_hbm, v_hbm, idx_hbm, out_hbm,
                  idxva, idxvb, karr, varr, qarr, oarr, a_arr, sem_k, sem_v):
    wid = lax.axis_index("s") * NC + lax.axis_index("c")
    base = wid * PER_W
    iota = lax.iota(jnp.int32, LANES)
    kf, vf, qf, of = karr, varr, qarr, oarr
    k2, v2, q2, o2 = karr, varr, qarr, oarr
    qrow = iota
    krows = [iota * N + n for n in range(N)]

    def do_chunk(ci, idxv_cur, idxv_next):
        t0 = base + ci * T
        pltpu.sync_copy(q_hbm.at[pl.ds(t0, T)], q2)
        pltpu.make_async_copy(k_hbm.at[idxv_cur], k2, sem_k).wait()
        cp_v = pltpu.async_copy(v_hbm.at[idxv_cur], v2, sem_v)

        for h in range(H):
            col0 = jnp.full((LANES,), h * DH, jnp.int32)

            def dbody(d, carry):
                colv, accs = carry
                qv = plsc.load_gather(qf, [qrow, colv])
                accs = [accs[n] + qv * plsc.load_gather(kf, [krows[n], colv])
                        for n in range(N)]
                return (colv + 1, accs)

            _, accs = lax.fori_loop(
                0, DH, dbody,
                (col0, [jnp.zeros((LANES,), jnp.float32)] * N))
            ps = [a * _SCALE for a in accs]
            m = ps[0]
            for n in range(1, N):
                m = jnp.maximum(m, ps[n])
            es = [jnp.exp(p - m) for p in ps]
            s = es[0]
            for n in range(1, N):
                s = s + es[n]
            for n in range(N):
                a_arr[h, n, :] = es[n] / s

        @pl.when(ci + 1 < CHUNKS)
        def _():
            t1 = base + (ci + 1) * T
            pltpu.sync_copy(idx_hbm.at[pl.ds(t1 * N, T * N)], idxv_next)
            pltpu.async_copy(k_hbm.at[idxv_next], k2, sem_k)

        pltpu.make_async_copy(v_hbm.at[idxv_cur], v2, sem_v).wait()

        for h in range(H):
            avs = [a_arr[h, n, :] for n in range(N)]
            col0 = jnp.full((LANES,), h * DH, jnp.int32)

            def cbody(d, colv):
                acc = avs[0] * plsc.load_gather(vf, [krows[0], colv])
                for n in range(1, N):
                    acc = acc + avs[n] * plsc.load_gather(vf, [krows[n], colv])
                plsc.store_scatter(of, [qrow, colv], acc)
                return colv + 1

            lax.fori_loop(0, DH, cbody, col0)

        pltpu.sync_copy(o2, out_hbm.at[pl.ds(t0, T)])

    pltpu.sync_copy(idx_hbm.at[pl.ds(base * N, T * N)], idxva)
    pltpu.async_copy(k_hbm.at[idxva], k2, sem_k)

    def pair_body(j, carry):
        do_chunk(2 * j, idxva, idxvb)
        do_chunk(2 * j + 1, idxvb, idxva)
        return carry

    lax.fori_loop(0, CHUNKS // 2, pair_body, 0)


def _attn_sc(q, k, v, qidx):
    mesh = plsc.VectorSubcoreMesh(core_axis_name="c", subcore_axis_name="s")
    fn = functools.partial(
        pl.kernel,
        mesh=mesh,
        out_type=jax.ShapeDtypeStruct((TOT, C), jnp.float32),
        scratch_types=[
            pltpu.VMEM((T * N,), jnp.int32),
            pltpu.VMEM((T * N,), jnp.int32),
            pltpu.VMEM((T * N, C), jnp.float32),
            pltpu.VMEM((T * N, C), jnp.float32),
            pltpu.VMEM((T, C), jnp.float32),
            pltpu.VMEM((T, C), jnp.float32),
            pltpu.VMEM((H, N, LANES), jnp.float32),
            pltpu.SemaphoreType.DMA,
            pltpu.SemaphoreType.DMA,
        ],
        compiler_params=pltpu.CompilerParams(use_tc_tiling_on_sc=False,
                                             needs_layout_passes=False),
    )(_attn_sc_body)
    return fn(q, k, v, qidx)


def kernel(x0, query, Wq, Wk, Wv, Wm, W1, W2, g1, b1, g2, b2):
    x0f = x0.reshape(TOT, C)
    qidx = (query.astype(jnp.int32)
            + (jnp.arange(B, dtype=jnp.int32) * L)[:, None, None]).reshape(-1)
    q, k, v = _qkv_tc(x0f, Wq, Wk, Wv, g1, b1)
    qvals = _attn_sc(q, k, v, qidx)
    out = _post_tc(x0f, qvals, Wm, W1, W2, g2, b2)
    return out.reshape(B, L, C)


# speedup vs baseline: 14.7958x; 3.5992x over previous
import functools
import math

import jax
import jax.numpy as jnp
from jax import lax
from jax.experimental import pallas as pl
from jax.experimental.pallas import tpu as pltpu
from jax.experimental.pallas import tpu_sc as plsc

B, L, C, N, H = 2, 4096, 256, 8, 8
DH = C // H
TOT = B * L
NC, NS, LANES = 2, 16, 16
NW = NC * NS
PER_W = TOT // NW
T = 16
CHUNKS = PER_W // T

_BLK = 512


def _dotT(a, w):
    return lax.dot_general(a, w, (((1,), (1,)), ((), ())),
                           preferred_element_type=jnp.float32)


def _ln(x, g, b, eps=1e-5):
    mu = jnp.mean(x, axis=-1, keepdims=True)
    xc = x - mu
    var = jnp.mean(xc * xc, axis=-1, keepdims=True)
    return xc * lax.rsqrt(var + eps) * g + b


def _qkv_body(x_ref, wq_ref, wk_ref, wv_ref, g_ref, b_ref, q_ref, k_ref, v_ref):
    xn = _ln(x_ref[...], g_ref[...], b_ref[...])
    q_ref[...] = _dotT(xn, wq_ref[...])
    k_ref[...] = _dotT(xn, wk_ref[...])
    v_ref[...] = _dotT(xn, wv_ref[...])


def _qkv_tc(x0f, Wq, Wk, Wv, g1, b1):
    grid = (TOT // _BLK,)
    row_spec = pl.BlockSpec((_BLK, C), lambda i: (i, 0))
    full_spec = pl.BlockSpec((C, C), lambda i: (0, 0))
    vec_spec = pl.BlockSpec((1, C), lambda i: (0, 0))
    return pl.pallas_call(
        _qkv_body,
        grid=grid,
        in_specs=[row_spec, full_spec, full_spec, full_spec, vec_spec, vec_spec],
        out_specs=[row_spec, row_spec, row_spec],
        out_shape=[jax.ShapeDtypeStruct((TOT, C), jnp.float32)] * 3,
    )(x0f, Wq, Wk, Wv, g1.reshape(1, C), b1.reshape(1, C))


def _post_body(x_ref, qv_ref, wm_ref, w1_ref, w2_ref, g_ref, b_ref, o_ref):
    msg = x_ref[...] + _dotT(qv_ref[...], wm_ref[...])
    hid = jnp.maximum(_dotT(msg, w1_ref[...]), 0.0)
    mlp = _dotT(hid, w2_ref[...])
    o_ref[...] = x_ref[...] + _ln(mlp, g_ref[...], b_ref[...])


def _post_tc(x0f, qvals, Wm, W1, W2, g2, b2):
    grid = (TOT // _BLK,)
    row_spec = pl.BlockSpec((_BLK, C), lambda i: (i, 0))
    full_spec = pl.BlockSpec((C, C), lambda i: (0, 0))
    vec_spec = pl.BlockSpec((1, C), lambda i: (0, 0))
    return pl.pallas_call(
        _post_body,
        grid=grid,
        in_specs=[row_spec, row_spec, full_spec, full_spec, full_spec,
                  vec_spec, vec_spec],
        out_specs=row_spec,
        out_shape=jax.ShapeDtypeStruct((TOT, C), jnp.float32),
    )(x0f, qvals, Wm, W1, W2, g2.reshape(1, C), b2.reshape(1, C))


_SCALE = 1.0 / math.sqrt(DH)


def _attn_sc_body(q_hbm, k_hbm, v_hbm, idx_hbm, out_hbm,
                  idxva, idxvb, karr, varr, qarr, oarr, a_arr, sem_k, sem_v):
    wid = lax.axis_index("s") * NC + lax.axis_index("c")
    base = wid * PER_W
    iota = lax.iota(jnp.int32, LANES)
    kf, vf, qf, of = karr, varr, qarr, oarr
    k2, v2, q2, o2 = karr, varr, qarr, oarr
    qrow = iota
    krows = [iota * N + n for n in range(N)]

    def do_chunk(ci, idxv_cur, idxv_next):
        t0 = base + ci * T
        pltpu.sync_copy(q_hbm.at[pl.ds(t0, T)], q2)
        pltpu.make_async_copy(k_hbm.at[idxv_cur], k2, sem_k).wait()
        cp_v = pltpu.async_copy(v_hbm.at[idxv_cur], v2, sem_v)

        for h in range(H):
            def dbody(d, accs):
                colv = ((iota + d) & (DH - 1)) | (h * DH)
                qv = plsc.load_gather(qf, [qrow, colv])
                return [accs[n] + qv * plsc.load_gather(kf, [krows[n], colv])
                        for n in range(N)]

            accs = lax.fori_loop(
                0, DH, dbody, [jnp.zeros((LANES,), jnp.float32)] * N)
            ps = [a * _SCALE for a in accs]
            m = ps[0]
            for n in range(1, N):
                m = jnp.maximum(m, ps[n])
            es = [jnp.exp(p - m) for p in ps]
            s = es[0]
            for n in range(1, N):
                s = s + es[n]
            for n in range(N):
                a_arr[h, n, :] = es[n] / s

        @pl.when(ci + 1 < CHUNKS)
        def _():
            t1 = base + (ci + 1) * T
            pltpu.sync_copy(idx_hbm.at[pl.ds(t1 * N, T * N)], idxv_next)
            pltpu.async_copy(k_hbm.at[idxv_next], k2, sem_k)

        pltpu.make_async_copy(v_hbm.at[idxv_cur], v2, sem_v).wait()

        for h in range(H):
            avs = [a_arr[h, n, :] for n in range(N)]

            def cbody(d, carry2):
                colv = ((iota + d) & (DH - 1)) | (h * DH)
                acc = avs[0] * plsc.load_gather(vf, [krows[0], colv])
                for n in range(1, N):
                    acc = acc + avs[n] * plsc.load_gather(vf, [krows[n], colv])
                plsc.store_scatter(of, [qrow, colv], acc)
                return carry2

            lax.fori_loop(0, DH, cbody, 0)

        pltpu.sync_copy(o2, out_hbm.at[pl.ds(t0, T)])

    pltpu.sync_copy(idx_hbm.at[pl.ds(base * N, T * N)], idxva)
    pltpu.async_copy(k_hbm.at[idxva], k2, sem_k)

    def pair_body(j, carry):
        do_chunk(2 * j, idxva, idxvb)
        do_chunk(2 * j + 1, idxvb, idxva)
        return carry

    lax.fori_loop(0, CHUNKS // 2, pair_body, 0)


def _attn_sc(q, k, v, qidx):
    mesh = plsc.VectorSubcoreMesh(core_axis_name="c", subcore_axis_name="s")
    fn = functools.partial(
        pl.kernel,
        mesh=mesh,
        out_type=jax.ShapeDtypeStruct((TOT, C), jnp.float32),
        scratch_types=[
            pltpu.VMEM((T * N,), jnp.int32),
            pltpu.VMEM((T * N,), jnp.int32),
            pltpu.VMEM((T * N, C), jnp.float32),
            pltpu.VMEM((T * N, C), jnp.float32),
            pltpu.VMEM((T, C), jnp.float32),
            pltpu.VMEM((T, C), jnp.float32),
            pltpu.VMEM((H, N, LANES), jnp.float32),
            pltpu.SemaphoreType.DMA,
            pltpu.SemaphoreType.DMA,
        ],
        compiler_params=pltpu.CompilerParams(use_tc_tiling_on_sc=False,
                                             needs_layout_passes=False),
    )(_attn_sc_body)
    return fn(q, k, v, qidx)


def kernel(x0, query, Wq, Wk, Wv, Wm, W1, W2, g1, b1, g2, b2):
    x0f = x0.reshape(TOT, C)
    qidx = (query.astype(jnp.int32)
            + (jnp.arange(B, dtype=jnp.int32) * L)[:, None, None]).reshape(-1)
    q, k, v = _qkv_tc(x0f, Wq, Wk, Wv, g1, b1)
    qvals = _attn_sc(q, k, v, qidx)
    out = _post_tc(x0f, qvals, Wm, W1, W2, g2, b2)
    return out.reshape(B, L, C)
